# Initial kernel scaffold; baseline (speedup 1.0000x reference)
import jax
import jax.numpy as jnp
from jax.experimental import pallas as pl


def kernel(hidden_states, attention_mask, position_ids, ln1_w, ln2_w, Wq, Wk, Wv, Wo, gate_w, W1, W3, W2):
    raise NotImplementedError("write your pallas kernel here")


# speedup vs baseline: 1.1798x; 1.1798x over previous
import functools
import math

import jax
import jax.numpy as jnp
from jax import lax
from jax.experimental import pallas as pl
from jax.experimental.pallas import tpu as pltpu
from jax.experimental.pallas import tpu_sc as plsc

B, S, D = 1, 2048, 1024
H, KVH, HD = 16, 8, 64
E, TOPK, F = 8, 2, 2048
EPS = 1e-6

NA = S * TOPK
BLK = 256
NBLK = NA // BLK + E
NPAD = NBLK * BLK
BQ = 256

NC, NS = 2, 16
NW = NC * NS


def _rope_apply(x, cos, sin, nheads):
    outs = []
    for h in range(nheads):
        xh = x[:, h * HD:(h + 1) * HD]
        xr = jnp.concatenate([-xh[:, HD // 2:], xh[:, :HD // 2]], axis=1)
        outs.append(xh * cos + xr * sin)
    return jnp.concatenate(outs, axis=1)


def _qkv_body(x_ref, ln1_ref, wq_ref, wk_ref, wv_ref, cos_ref, sin_ref,
              q_ref, k_ref, v_ref):
    x = x_ref[...]
    r = lax.rsqrt(jnp.mean(x * x, axis=1, keepdims=True) + EPS)
    h = (x * r * ln1_ref[...]).astype(jnp.bfloat16)
    cos = cos_ref[...]
    sin = sin_ref[...]
    q = jnp.dot(h, wq_ref[...].astype(jnp.bfloat16),
                preferred_element_type=jnp.float32)
    q_ref[...] = _rope_apply(q, cos, sin, H).astype(jnp.bfloat16)
    k = jnp.dot(h, wk_ref[...].astype(jnp.bfloat16),
                preferred_element_type=jnp.float32)
    k_ref[...] = _rope_apply(k, cos, sin, KVH).astype(jnp.bfloat16)
    v_ref[...] = jnp.dot(h, wv_ref[...].astype(jnp.bfloat16),
                         preferred_element_type=jnp.float32).astype(jnp.bfloat16)


def _qkv_call(x, ln1_w, Wq, Wk, Wv, cos, sin):
    nb = S // BQ
    return pl.pallas_call(
        _qkv_body,
        grid=(nb,),
        in_specs=[
            pl.BlockSpec((BQ, D), lambda i: (i, 0)),
            pl.BlockSpec((1, D), lambda i: (0, 0)),
            pl.BlockSpec((D, H * HD), lambda i: (0, 0)),
            pl.BlockSpec((D, KVH * HD), lambda i: (0, 0)),
            pl.BlockSpec((D, KVH * HD), lambda i: (0, 0)),
            pl.BlockSpec((BQ, HD), lambda i: (i, 0)),
            pl.BlockSpec((BQ, HD), lambda i: (i, 0)),
        ],
        out_specs=[
            pl.BlockSpec((BQ, H * HD), lambda i: (i, 0)),
            pl.BlockSpec((BQ, KVH * HD), lambda i: (i, 0)),
            pl.BlockSpec((BQ, KVH * HD), lambda i: (i, 0)),
        ],
        out_shape=[
            jax.ShapeDtypeStruct((S, H * HD), jnp.bfloat16),
            jax.ShapeDtypeStruct((S, KVH * HD), jnp.bfloat16),
            jax.ShapeDtypeStruct((S, KVH * HD), jnp.bfloat16),
        ],
    )(x, ln1_w.reshape(1, D), Wq, Wk, Wv, cos, sin)


def _attn_body(q_ref, k_ref, v_ref, o_ref):
    qb = q_ref[0]
    s = lax.dot_general(qb, k_ref[0], (((1,), (1,)), ((), ())),
                        preferred_element_type=jnp.float32)
    s = s * (1.0 / math.sqrt(HD))
    i = pl.program_id(1)
    row = i * BQ + lax.broadcasted_iota(jnp.int32, (BQ, S), 0)
    col = lax.broadcasted_iota(jnp.int32, (BQ, S), 1)
    s = jnp.where(col <= row, s, -1e30)
    m = jnp.max(s, axis=1, keepdims=True)
    e = jnp.exp(s - m)
    l = jnp.sum(e, axis=1, keepdims=True)
    p = (e / l).astype(jnp.bfloat16)
    o_ref[0] = jnp.dot(p, v_ref[0],
                       preferred_element_type=jnp.float32).astype(jnp.bfloat16)


def _attn_call(q3, k3, v3):
    return pl.pallas_call(
        _attn_body,
        grid=(H, S // BQ),
        in_specs=[
            pl.BlockSpec((1, BQ, HD), lambda h, i: (h, i, 0)),
            pl.BlockSpec((1, S, HD), lambda h, i: (h // 2, 0, 0)),
            pl.BlockSpec((1, S, HD), lambda h, i: (h // 2, 0, 0)),
        ],
        out_specs=pl.BlockSpec((1, BQ, HD), lambda h, i: (h, i, 0)),
        out_shape=jax.ShapeDtypeStruct((H, S, HD), jnp.bfloat16),
    )(q3, k3, v3)


def _post_body(a_ref, res_ref, wo_ref, ln2_ref, gw_ref,
               x_ref, h2_ref, lg_ref, s0_ref, s1_ref, w0_ref, w1_ref):
    ao = jnp.dot(a_ref[...], wo_ref[...].astype(jnp.bfloat16),
                 preferred_element_type=jnp.float32)
    x = res_ref[...] + ao
    x_ref[...] = x
    r = lax.rsqrt(jnp.mean(x * x, axis=1, keepdims=True) + EPS)
    h2 = x * r * ln2_ref[...]
    h2_ref[...] = h2
    lg = jnp.dot(h2, gw_ref[...], preferred_element_type=jnp.float32)
    lg_ref[...] = lg
    iota8 = lax.broadcasted_iota(jnp.int32, (BQ, E), 1)
    m1 = jnp.max(lg, axis=1, keepdims=True)
    a1 = jnp.min(jnp.where(lg == m1, iota8, E), axis=1, keepdims=True)
    lm = jnp.where(iota8 == a1, -1e30, lg)
    m2 = jnp.max(lm, axis=1, keepdims=True)
    a2 = jnp.min(jnp.where(lm == m2, iota8, E), axis=1, keepdims=True)
    p1 = 1.0 / (1.0 + jnp.exp(m2 - m1))
    s0_ref[...] = a1[:, 0]
    s1_ref[...] = a2[:, 0]
    w0_ref[...] = p1[:, 0]
    w1_ref[...] = 1.0 - p1[:, 0]


def _post_call(attn, resid, Wo, ln2_w, gate_w):
    nb = S // BQ
    return pl.pallas_call(
        _post_body,
        grid=(nb,),
        in_specs=[
            pl.BlockSpec((BQ, H * HD), lambda i: (i, 0)),
            pl.BlockSpec((BQ, D), lambda i: (i, 0)),
            pl.BlockSpec((H * HD, D), lambda i: (0, 0)),
            pl.BlockSpec((1, D), lambda i: (0, 0)),
            pl.BlockSpec((D, E), lambda i: (0, 0)),
        ],
        out_specs=[
            pl.BlockSpec((BQ, D), lambda i: (i, 0)),
            pl.BlockSpec((BQ, D), lambda i: (i, 0)),
            pl.BlockSpec((BQ, E), lambda i: (i, 0)),
            pl.BlockSpec((BQ,), lambda i: (i,)),
            pl.BlockSpec((BQ,), lambda i: (i,)),
            pl.BlockSpec((BQ,), lambda i: (i,)),
            pl.BlockSpec((BQ,), lambda i: (i,)),
        ],
        out_shape=[
            jax.ShapeDtypeStruct((S, D), jnp.float32),
            jax.ShapeDtypeStruct((S, D), jnp.float32),
            jax.ShapeDtypeStruct((S, E), jnp.float32),
            jax.ShapeDtypeStruct((S,), jnp.int32),
            jax.ShapeDtypeStruct((S,), jnp.int32),
            jax.ShapeDtypeStruct((S,), jnp.float32),
            jax.ShapeDtypeStruct((S,), jnp.float32),
        ],
    )(attn, resid, Wo, ln2_w.reshape(1, D), gate_w)


def _route_body(s0_h, s1_h, inv_h, src_h, bexp_h, selv, posv, srcv, bexpv):
    cid = lax.axis_index("c")
    sid = lax.axis_index("s")

    @pl.when(jnp.logical_and(cid == 0, sid == 0))
    def _():
        pltpu.sync_copy(s0_h, selv.at[pl.ds(0, S)])
        pltpu.sync_copy(s1_h, selv.at[pl.ds(S, S)])
        lane = lax.iota(jnp.int32, 16)
        zero16 = jnp.zeros((16,), jnp.int32)
        one16 = jnp.ones((16,), jnp.int32)
        lgblk = BLK.bit_length() - 1

        def h1(g, cnt):
            ev = selv[pl.ds(g * 16, 16)]
            for e in range(E):
                pope = jnp.full((16,), jnp.sum((ev == e).astype(jnp.int32)))
                cnt = cnt + jnp.where(lane == e, pope, zero16)
            return cnt

        cnt = lax.fori_loop(0, NA // 16, h1, zero16)
        padded = lax.shift_left(
            lax.shift_right_logical(cnt + (BLK - 1), lgblk), lgblk)
        csum = plsc.cumsum(padded)
        base = csum - padded
        cb = lax.shift_right_logical(base, lgblk)
        for vg in range(2):
            bid = lane + vg * 16
            acc = jnp.zeros((16,), jnp.int32)
            for e in range(E):
                cbe = jnp.full((16,), jnp.sum(jnp.where(lane == e, cb, zero16)))
                acc = acc + (cbe <= bid).astype(jnp.int32)
            bexpv[pl.ds(vg * 16, 16)] = acc - 1

        def zb(g, c):
            srcv[pl.ds(g * 16, 16)] = zero16
            return c

        lax.fori_loop(0, NPAD // 16, zb, 0)

        def p2(g, cntv):
            ev = selv[pl.ds(g * 16, 16)]
            gv = jnp.full((16,), g * 16)
            tok = jnp.bitwise_and(lane + gv, S - 1)
            posvec = zero16
            for e in range(E):
                me = ev == e
                cs = plsc.cumsum(jnp.where(me, one16, zero16))
                bs = jnp.full((16,), jnp.sum(jnp.where(lane == e, cntv, zero16)))
                posvec = jnp.where(me, bs + cs - 1, posvec)
                pope = jnp.full((16,), jnp.sum(me.astype(jnp.int32)))
                cntv = cntv + jnp.where(lane == e, pope, zero16)
            posv[pl.ds(g * 16, 16)] = posvec
            plsc.store_scatter(srcv, [posvec], tok)
            return cntv

        lax.fori_loop(0, NA // 16, p2, base)
        pltpu.sync_copy(posv, inv_h)
        pltpu.sync_copy(srcv, src_h)
        pltpu.sync_copy(bexpv, bexp_h)


def _route_call(sel0, sel1):
    mesh = plsc.VectorSubcoreMesh(core_axis_name="c", subcore_axis_name="s")
    return pl.kernel(
        _route_body,
        out_type=[
            jax.ShapeDtypeStruct((NA,), jnp.int32),
            jax.ShapeDtypeStruct((NPAD,), jnp.int32),
            jax.ShapeDtypeStruct((32,), jnp.int32),
        ],
        mesh=mesh,
        compiler_params=pltpu.CompilerParams(needs_layout_passes=False),
        scratch_types=[
            pltpu.VMEM((NA,), jnp.int32),
            pltpu.VMEM((NA,), jnp.int32),
            pltpu.VMEM((NPAD,), jnp.int32),
            pltpu.VMEM((32,), jnp.int32),
        ],
    )(sel0, sel1)


_G_CHUNK = 64
_G_PER_W = NPAD // NW


def _gather_body(src_h, h2_h, x_h, idxv, rowsv, sem):
    wid = lax.axis_index("s") * NC + lax.axis_index("c")
    base = wid * _G_PER_W
    for c in range(_G_PER_W // _G_CHUNK):
        pltpu.sync_copy(src_h.at[pl.ds(base + c * _G_CHUNK, _G_CHUNK)], idxv)
        pltpu.async_copy(h2_h.at[idxv], rowsv, sem).wait()
        pltpu.sync_copy(rowsv, x_h.at[pl.ds(base + c * _G_CHUNK, _G_CHUNK)])


def _gather_call(src, h2):
    mesh = plsc.VectorSubcoreMesh(core_axis_name="c", subcore_axis_name="s")
    return pl.kernel(
        _gather_body,
        out_type=jax.ShapeDtypeStruct((NPAD, D), jnp.float32),
        mesh=mesh,
        compiler_params=pltpu.CompilerParams(needs_layout_passes=False),
        scratch_types=[
            pltpu.VMEM((_G_CHUNK,), jnp.int32),
            pltpu.VMEM((_G_CHUNK, D), jnp.float32),
            pltpu.SemaphoreType.DMA,
        ],
    )(src, h2)


def _ffn_body(bexp_ref, x_ref, w1_ref, w3_ref, w2_ref, y_ref):
    xb = x_ref[...].astype(jnp.bfloat16)
    a = jnp.dot(xb, w1_ref[0].astype(jnp.bfloat16),
                preferred_element_type=jnp.float32)
    b = jnp.dot(xb, w3_ref[0].astype(jnp.bfloat16),
                preferred_element_type=jnp.float32)
    hcur = (a * (1.0 / (1.0 + jnp.exp(-a))) * b).astype(jnp.bfloat16)
    y_ref[...] = jnp.dot(hcur, w2_ref[0].astype(jnp.bfloat16),
                         preferred_element_type=jnp.float32)


def _ffn_call(bexp, X, W1, W3, W2):
    return pl.pallas_call(
        _ffn_body,
        grid_spec=pltpu.PrefetchScalarGridSpec(
            num_scalar_prefetch=1,
            grid=(NBLK,),
            in_specs=[
                pl.BlockSpec((BLK, D), lambda i, be: (i, 0)),
                pl.BlockSpec((1, D, F), lambda i, be: (be[i], 0, 0)),
                pl.BlockSpec((1, D, F), lambda i, be: (be[i], 0, 0)),
                pl.BlockSpec((1, F, D), lambda i, be: (be[i], 0, 0)),
            ],
            out_specs=pl.BlockSpec((BLK, D), lambda i, be: (i, 0)),
        ),
        out_shape=jax.ShapeDtypeStruct((NPAD, D), jnp.float32),
    )(bexp, X, W1, W3, W2)


_M_PER_W = S // NW
_M_CHUNK = 16


def _comb_body(xres_h, y_h, inv_h, w0_h, w1_h, out_h,
               iv0v, iv1v, w0v, w1v, xbuf, y0buf, y1buf, sem0, sem1):
    wid = lax.axis_index("s") * NC + lax.axis_index("c")
    t0 = wid * _M_PER_W
    pltpu.sync_copy(inv_h.at[pl.ds(t0, _M_PER_W)], iv0v)
    pltpu.sync_copy(inv_h.at[pl.ds(S + t0, _M_PER_W)], iv1v)
    pltpu.sync_copy(w0_h.at[pl.ds(t0, _M_PER_W)], w0v)
    pltpu.sync_copy(w1_h.at[pl.ds(t0, _M_PER_W)], w1v)
    for c in range(_M_PER_W // _M_CHUNK):
        i0 = iv0v[pl.ds(c * _M_CHUNK, 16)]
        i1 = iv1v[pl.ds(c * _M_CHUNK, 16)]
        cp0 = pltpu.async_copy(y_h.at[i0], y0buf, sem0)
        cp1 = pltpu.async_copy(y_h.at[i1], y1buf, sem1)
        pltpu.sync_copy(xres_h.at[pl.ds(t0 + c * _M_CHUNK, _M_CHUNK)], xbuf)
        cp0.wait()
        cp1.wait()

        wv0 = w0v[pl.ds(c * _M_CHUNK, 16)]
        wv1 = w1v[pl.ds(c * _M_CHUNK, 16)]
        for rr in range(_M_CHUNK):
            wa = wv0[rr]
            wb = wv1[rr]

            def col(cc, carry, rr=rr, wa=wa, wb=wb):
                sl = pl.ds(cc * 16, 16)
                xbuf[rr, sl] = (xbuf[rr, sl] + wa * y0buf[rr, sl]
                                + wb * y1buf[rr, sl])
                return carry

            lax.fori_loop(0, D // 16, col, 0)
        pltpu.sync_copy(xbuf, out_h.at[pl.ds(t0 + c * _M_CHUNK, _M_CHUNK)])


def _comb_call(xres, Y, inv, w0, w1):
    mesh = plsc.VectorSubcoreMesh(core_axis_name="c", subcore_axis_name="s")
    return pl.kernel(
        _comb_body,
        out_type=jax.ShapeDtypeStruct((S, D), jnp.float32),
        mesh=mesh,
        compiler_params=pltpu.CompilerParams(needs_layout_passes=False),
        scratch_types=[
            pltpu.VMEM((_M_PER_W,), jnp.int32),
            pltpu.VMEM((_M_PER_W,), jnp.int32),
            pltpu.VMEM((_M_PER_W,), jnp.float32),
            pltpu.VMEM((_M_PER_W,), jnp.float32),
            pltpu.VMEM((_M_CHUNK, D), jnp.float32),
            pltpu.VMEM((_M_CHUNK, D), jnp.float32),
            pltpu.VMEM((_M_CHUNK, D), jnp.float32),
            pltpu.SemaphoreType.DMA,
            pltpu.SemaphoreType.DMA,
        ],
    )(xres, Y, inv, w0, w1)


def kernel(hidden_states, attention_mask, position_ids, ln1_w, ln2_w,
           Wq, Wk, Wv, Wo, gate_w, W1, W3, W2):
    x = hidden_states.reshape(S, D)
    pos = position_ids.reshape(S).astype(jnp.float32)
    inv_freq = 1.0 / (10000.0 ** (jnp.arange(0, HD, 2, dtype=jnp.float32) / HD))
    freqs = pos[:, None] * inv_freq
    emb = jnp.concatenate([freqs, freqs], axis=-1)
    cos = jnp.cos(emb)
    sin = jnp.sin(emb)

    q, k, v = _qkv_call(x, ln1_w, Wq, Wk, Wv, cos, sin)
    q3 = q.reshape(S, H, HD).transpose(1, 0, 2)
    k3 = k.reshape(S, KVH, HD).transpose(1, 0, 2)
    v3 = v.reshape(S, KVH, HD).transpose(1, 0, 2)
    attn3 = _attn_call(q3, k3, v3)
    attn = attn3.transpose(1, 0, 2).reshape(S, H * HD)
    xres, h2, logits, sel0, sel1, w0, w1 = _post_call(attn, x, Wo, ln2_w, gate_w)
    inv, src, bexp = _route_call(sel0, sel1)
    X = _gather_call(src, h2)
    Y = _ffn_call(bexp, X, W1, W3, W2)
    xfin = _comb_call(xres, Y, inv, w0, w1)
    return xfin.reshape(B, S, D), logits.reshape(B, S, E)
